# baseline (speedup 1.0000x reference)
import functools

import jax
import jax.numpy as jnp
from jax import lax
from jax.experimental import pallas as pl
from jax.experimental.pallas import tpu as pltpu
from jax.experimental.pallas import tpu_sc as plsc

NC = 2
NS = 16
LANES = 16


def _bmm_halves(a, w, bn):
    n, k = a.shape
    g, _, o = w.shape
    half = o // 2

    def body(a_ref, w_ref, oa_ref, ob_ref):
        acc = jnp.dot(a_ref[...], w_ref[0], preferred_element_type=jnp.float32)
        oa_ref[0] = acc[:, :half]
        ob_ref[0] = acc[:, half:]

    return pl.pallas_call(
        body,
        grid=(g, n // bn),
        in_specs=[
            pl.BlockSpec((bn, k), lambda gi, ni: (ni, 0)),
            pl.BlockSpec((1, k, o), lambda gi, ni: (gi, 0, 0)),
        ],
        out_specs=[
            pl.BlockSpec((1, bn, half), lambda gi, ni: (gi, ni, 0)),
            pl.BlockSpec((1, bn, half), lambda gi, ni: (gi, ni, 0)),
        ],
        out_shape=[
            jax.ShapeDtypeStruct((g, n, half), jnp.float32),
            jax.ShapeDtypeStruct((g, n, half), jnp.float32),
        ],
    )(a, w)


def _relu_add(agga, aggb, hra, hrb, bias, bn):
    n, half = agga.shape

    def body(aa, ab, ra, rb, b_ref, h_ref):
        b = b_ref[...]
        xa = jnp.maximum(aa[...] + ra[...] + b[:, :half], 0.0)
        xb = jnp.maximum(ab[...] + rb[...] + b[:, half:], 0.0)
        h_ref[...] = jnp.concatenate([xa, xb], axis=1)

    return pl.pallas_call(
        body,
        grid=(n // bn,),
        in_specs=[
            pl.BlockSpec((bn, half), lambda i: (i, 0)),
            pl.BlockSpec((bn, half), lambda i: (i, 0)),
            pl.BlockSpec((bn, half), lambda i: (i, 0)),
            pl.BlockSpec((bn, half), lambda i: (i, 0)),
            pl.BlockSpec((1, 2 * half), lambda i: (0, 0)),
        ],
        out_specs=pl.BlockSpec((bn, 2 * half), lambda i: (i, 0)),
        out_shape=jax.ShapeDtypeStruct((n, 2 * half), jnp.float32),
    )(agga, aggb, hra, hrb, bias)


def _fc(hsel, fcwp, fcbp):
    m, _ = hsel.shape

    def body(h_ref, w_ref, b_ref, o_ref):
        o_ref[...] = (
            jnp.dot(h_ref[...], w_ref[...], preferred_element_type=jnp.float32)
            + b_ref[...]
        )

    return pl.pallas_call(
        body,
        out_shape=jax.ShapeDtypeStruct((m, 128), jnp.float32),
    )(hsel, fcwp, fcbp)


def _edge_prep(src, dst, et, n_nodes, num_rel):
    e = src.shape[0]
    nr = n_nodes * num_rel
    epw = e // (NC * NS)
    ck = 8000
    cn = 1000
    mesh = plsc.VectorSubcoreMesh(core_axis_name="c", subcore_axis_name="s")

    @functools.partial(
        pl.kernel,
        out_type=[
            jax.ShapeDtypeStruct((e,), jnp.int32),
            jax.ShapeDtypeStruct((e,), jnp.float32),
        ],
        mesh=mesh,
        compiler_params=pltpu.CompilerParams(needs_layout_passes=False),
        scratch_types=[
            pltpu.VMEM((nr,), jnp.float32),
            pltpu.VMEM((ck,), jnp.int32),
            pltpu.VMEM((ck,), jnp.int32),
            pltpu.VMEM((cn,), jnp.int32),
            pltpu.VMEM((cn,), jnp.int32),
            pltpu.VMEM((cn,), jnp.int32),
            pltpu.VMEM((cn,), jnp.int32),
            pltpu.VMEM((cn,), jnp.float32),
        ],
    )
    def k(src_hbm, dst_hbm, et_hbm, rid_hbm, norm_hbm,
          counts_v, dc_v, ec_v, s2_v, d2_v, e2_v, ridc_v, normc_v):
        c = lax.axis_index("c")
        s = lax.axis_index("s")
        wid = s * NC + c
        one16 = jnp.ones((LANES,), jnp.float32)
        zero16 = jnp.zeros((LANES,), jnp.float32)

        @pl.loop(0, nr // LANES)
        def _(i):
            counts_v[pl.ds(i * LANES, LANES)] = zero16

        @pl.loop(0, e // ck)
        def _(kc):
            base = kc * ck
            pltpu.sync_copy(dst_hbm.at[pl.ds(base, ck)], dc_v)
            pltpu.sync_copy(et_hbm.at[pl.ds(base, ck)], ec_v)

            @pl.loop(0, ck // LANES)
            def _(i):
                sl = pl.ds(i * LANES, LANES)
                seg = dc_v[sl] * num_rel + ec_v[sl]
                plsc.addupdate_scatter(counts_v, [seg], one16)

        w0 = wid * epw

        @pl.loop(0, epw // cn)
        def _(kc):
            base = w0 + kc * cn
            pltpu.sync_copy(src_hbm.at[pl.ds(base, cn)], s2_v)
            pltpu.sync_copy(dst_hbm.at[pl.ds(base, cn)], d2_v)
            pltpu.sync_copy(et_hbm.at[pl.ds(base, cn)], e2_v)

            def group(off):
                sl = pl.ds(off, LANES)
                d16 = d2_v[sl]
                e16 = e2_v[sl]
                seg = d16 * num_rel + e16
                cnt = plsc.load_gather(counts_v, [seg])
                ridc_v[sl] = e16 * n_nodes + s2_v[sl]
                normc_v[sl] = 1.0 / jnp.maximum(cnt, 1.0)

            @pl.loop(0, cn // LANES)
            def _(i):
                group(i * LANES)

            if cn % LANES:
                group(cn - LANES)

            pltpu.sync_copy(ridc_v, rid_hbm.at[pl.ds(base, cn)])
            pltpu.sync_copy(normc_v, norm_hbm.at[pl.ds(base, cn)])

    return k(src, dst, et)


def _msg(xra, xrb, rid2, dst2, norm2, zrows, n_nodes, half):
    rows, kk = rid2.shape
    rt = rows // NS
    rst = 624
    rww = n_nodes - rst * (NS - 1)
    mesh = plsc.VectorSubcoreMesh(core_axis_name="c", subcore_axis_name="s")

    sq = 16

    @functools.partial(
        pl.kernel,
        out_type=[jax.ShapeDtypeStruct((n_nodes, half), jnp.float32)] * 2,
        mesh=mesh,
        compiler_params=pltpu.CompilerParams(needs_layout_passes=False),
        scratch_types=[
            pltpu.VMEM_SHARED((n_nodes, half), jnp.float32),
            pltpu.VMEM((sq, kk), jnp.int32),
            pltpu.VMEM((sq, kk), jnp.int32),
            pltpu.VMEM((sq, kk), jnp.float32),
            pltpu.VMEM((kk, half), jnp.float32),
            pltpu.VMEM((kk, half), jnp.float32),
            pltpu.SemaphoreType.DMA,
            pltpu.SemaphoreType.DMA,
            pltpu.SemaphoreType.DMA,
            pltpu.SemaphoreType.DMA,
        ],
    )
    def k(xra_hbm, xrb_hbm, rid_hbm, dst_hbm, norm_hbm, z_hbm,
          outa_hbm, outb_hbm, agg_sh, rid_t, dst_t, norm_t,
          msg0, msg1, gsem0, gsem1, ssem0, ssem1):
        c = lax.axis_index("c")
        s = lax.axis_index("s")

        def half_body(xr_hbm, out_hbm):
            pltpu.sync_copy(z_hbm, agg_sh.at[pl.ds(s * rst, rww)])
            plsc.subcore_barrier()

            def gather(kc, buf, sem):
                pltpu.async_copy(xr_hbm.at[rid_t.at[kc]], buf, sem)

            def wait_dma(buf, sem):
                pltpu.make_async_copy(xr_hbm.at[rid_t.at[0]], buf, sem).wait()

            def scale(kc, buf):
                @pl.loop(0, kk // LANES)
                def _(g):
                    nvec = norm_t[kc, pl.ds(g * LANES, LANES)]
                    for i in range(LANES):
                        nv = nvec[i]
                        row = g * LANES + i
                        for j in range(half // LANES):
                            sl = pl.ds(j * LANES, LANES)
                            buf[row, sl] = buf[row, sl] * nv

            def scatter(kc, buf, sem):
                pltpu.async_copy(buf, agg_sh.at[dst_t.at[kc]], sem, add=True)

            @pl.loop(0, rt // sq)
            def _(q):
                r0 = s * rt + q * sq
                pltpu.sync_copy(rid_hbm.at[pl.ds(r0, sq)], rid_t)
                pltpu.sync_copy(dst_hbm.at[pl.ds(r0, sq)], dst_t)
                pltpu.sync_copy(norm_hbm.at[pl.ds(r0, sq)], norm_t)
                gather(0, msg0, gsem0)

                @pl.loop(0, sq // 2)
                def _(p):
                    k0 = 2 * p

                    @pl.when(p > 0)
                    def _():
                        wait_dma(msg1, ssem1)

                    gather(k0 + 1, msg1, gsem1)
                    wait_dma(msg0, gsem0)
                    scale(k0, msg0)
                    scatter(k0, msg0, ssem0)
                    wait_dma(msg1, gsem1)
                    scale(k0 + 1, msg1)
                    wait_dma(msg0, ssem0)

                    @pl.when(k0 + 2 < sq)
                    def _():
                        gather(k0 + 2, msg0, gsem0)

                    scatter(k0 + 1, msg1, ssem1)

                wait_dma(msg1, ssem1)

            plsc.subcore_barrier()
            pltpu.sync_copy(agg_sh.at[pl.ds(s * rst, rww)],
                            out_hbm.at[pl.ds(s * rst, rww)])

        @pl.when(c == 0)
        def _():
            half_body(xra_hbm, outa_hbm)

        @pl.when(c == 1)
        def _():
            half_body(xrb_hbm, outb_hbm)

    return k(xra, xrb, rid2, dst2, norm2, zrows)


def _edge_compact(tni, rid, dstn, norm, n_nodes, cap):
    e = rid.shape[0]
    b = tni.shape[0]
    nw = NC * NS
    epw = e // nw
    cn = 1000
    ecap = cap * nw
    mesh = plsc.VectorSubcoreMesh(core_axis_name="c", subcore_axis_name="s")

    @functools.partial(
        pl.kernel,
        out_type=[
            jax.ShapeDtypeStruct((ecap,), jnp.int32),
            jax.ShapeDtypeStruct((ecap,), jnp.int32),
            jax.ShapeDtypeStruct((ecap,), jnp.float32),
            jax.ShapeDtypeStruct((nw * LANES,), jnp.int32),
        ],
        mesh=mesh,
        compiler_params=pltpu.CompilerParams(needs_layout_passes=False),
        scratch_types=[
            pltpu.VMEM((n_nodes,), jnp.int32),
            pltpu.VMEM((b,), jnp.int32),
            pltpu.VMEM((cn,), jnp.int32),
            pltpu.VMEM((cn,), jnp.int32),
            pltpu.VMEM((cn,), jnp.float32),
            pltpu.VMEM((cap,), jnp.int32),
            pltpu.VMEM((cap,), jnp.int32),
            pltpu.VMEM((cap,), jnp.float32),
            pltpu.VMEM((LANES,), jnp.int32),
        ],
    )
    def k(tni_hbm, rid_hbm, dst_hbm, norm_hbm,
          crid_hbm, cdst_hbm, cnorm_hbm, ecnt_hbm,
          mark_v, tni_v, ridc_v, dstc_v, nrmc_v, cr_v, cd_v, cw_v, cnt_v):
        c = lax.axis_index("c")
        s = lax.axis_index("s")
        wid = s * NC + c
        izero16 = jnp.zeros((LANES,), jnp.int32)
        ione16 = jnp.ones((LANES,), jnp.int32)
        fzero16 = jnp.zeros((LANES,), jnp.float32)
        lane = lax.iota(jnp.int32, LANES)

        @pl.loop(0, n_nodes // LANES)
        def _(i):
            mark_v[pl.ds(i * LANES, LANES)] = izero16

        pltpu.sync_copy(tni_hbm, tni_v)

        @pl.loop(0, b // LANES)
        def _(g):
            t16 = tni_v[pl.ds(g * LANES, LANES)]
            plsc.store_scatter(mark_v, [t16], ione16)

        if b % LANES:
            t16 = tni_v[pl.ds(b - LANES, LANES)]
            plsc.store_scatter(mark_v, [t16], ione16,
                               mask=lane >= LANES - (b % LANES))

        @pl.loop(0, cap // LANES)
        def _(i):
            sl = pl.ds(i * LANES, LANES)
            cr_v[sl] = izero16
            cd_v[sl] = izero16
            cw_v[sl] = fzero16

        w0 = wid * epw

        def do_group(koff, cnt, mask_extra=None):
            sl = pl.ds(koff, LANES)
            d16 = dstc_v[sl]
            r16 = ridc_v[sl]
            n16 = nrmc_v[sl]
            m16 = plsc.load_gather(mark_v, [d16]) > 0
            if mask_extra is not None:
                m16 = m16 & mask_extra
            osl = pl.ds(cnt, LANES)
            plsc.store_compressed(cr_v.at[osl], r16, mask=m16)
            plsc.store_compressed(cd_v.at[osl], d16, mask=m16)
            plsc.store_compressed(cw_v.at[osl], n16, mask=m16)
            return cnt + plsc.all_reduce_population_count(m16)[0]

        @pl.loop(0, epw // cn, init_carry=jnp.int32(0))
        def total(kc, cnt):
            base = w0 + kc * cn
            pltpu.sync_copy(rid_hbm.at[pl.ds(base, cn)], ridc_v)
            pltpu.sync_copy(dst_hbm.at[pl.ds(base, cn)], dstc_v)
            pltpu.sync_copy(norm_hbm.at[pl.ds(base, cn)], nrmc_v)

            @pl.loop(0, cn // LANES, init_carry=cnt)
            def cnt2(g, cc):
                return do_group(g * LANES, cc)

            if cn % LANES:
                cnt2 = do_group(cn - LANES, cnt2,
                                mask_extra=lane >= LANES - (cn % LANES))
            return cnt2

        pltpu.sync_copy(cr_v, crid_hbm.at[pl.ds(wid * cap, cap)])
        pltpu.sync_copy(cd_v, cdst_hbm.at[pl.ds(wid * cap, cap)])
        pltpu.sync_copy(cw_v, cnorm_hbm.at[pl.ds(wid * cap, cap)])
        cnt_v[pl.ds(0, LANES)] = jnp.full((LANES,), total, jnp.int32)
        pltpu.sync_copy(cnt_v, ecnt_hbm.at[pl.ds(wid * LANES, LANES)])

    return k(tni, rid, dstn, norm)


def _msg_dyn(xra, xrb, crid, cdst, cnorm, ecnt, zrows, n_nodes, half, kk):
    nw = NC * NS
    cap = crid.shape[0] // nw
    rst = 624
    rww = n_nodes - rst * (NS - 1)
    mesh = plsc.VectorSubcoreMesh(core_axis_name="c", subcore_axis_name="s")

    @functools.partial(
        pl.kernel,
        out_type=[jax.ShapeDtypeStruct((n_nodes, half), jnp.float32)] * 2,
        mesh=mesh,
        compiler_params=pltpu.CompilerParams(needs_layout_passes=False),
        scratch_types=[
            pltpu.VMEM_SHARED((n_nodes, half), jnp.float32),
            pltpu.VMEM((kk,), jnp.int32),
            pltpu.VMEM((kk,), jnp.int32),
            pltpu.VMEM((kk,), jnp.float32),
            pltpu.VMEM((kk, half), jnp.float32),
            pltpu.VMEM((LANES,), jnp.int32),
            pltpu.SemaphoreType.DMA,
        ],
    )
    def k(xra_hbm, xrb_hbm, crid_hbm, cdst_hbm, cnorm_hbm, ecnt_hbm, z_hbm,
          outa_hbm, outb_hbm, agg_sh, rid_v, dst_v, nrm_v, msg_v, cnt_v, sem):
        c = lax.axis_index("c")
        s = lax.axis_index("s")

        def region(xr_hbm, w):
            pltpu.sync_copy(ecnt_hbm.at[pl.ds(w * LANES, LANES)], cnt_v)
            count = cnt_v[pl.ds(0, LANES)][0]

            @pl.loop(0, cap // kk)
            def _(kc):
                @pl.when(kc * kk < count)
                def _():
                    base = w * cap + kc * kk
                    pltpu.sync_copy(crid_hbm.at[pl.ds(base, kk)], rid_v)
                    pltpu.sync_copy(cdst_hbm.at[pl.ds(base, kk)], dst_v)
                    pltpu.sync_copy(cnorm_hbm.at[pl.ds(base, kk)], nrm_v)
                    pltpu.async_copy(xr_hbm.at[rid_v], msg_v, sem).wait()

                    @pl.loop(0, kk // LANES)
                    def _(g):
                        nvec = nrm_v[pl.ds(g * LANES, LANES)]
                        for i in range(LANES):
                            nv = nvec[i]
                            row = g * LANES + i
                            for j in range(half // LANES):
                                sl = pl.ds(j * LANES, LANES)
                                msg_v[row, sl] = msg_v[row, sl] * nv

                    pltpu.sync_copy(msg_v, agg_sh.at[dst_v], add=True)

        def half_body(xr_hbm, out_hbm):
            pltpu.sync_copy(z_hbm, agg_sh.at[pl.ds(s * rst, rww)])
            plsc.subcore_barrier()
            for wofs in range(NC):
                region(xr_hbm, s * NC + wofs)
            plsc.subcore_barrier()
            pltpu.sync_copy(agg_sh.at[pl.ds(s * rst, rww)],
                            out_hbm.at[pl.ds(s * rst, rww)])

        @pl.when(c == 0)
        def _():
            half_body(xra_hbm, outa_hbm)

        @pl.when(c == 1)
        def _():
            half_body(xrb_hbm, outb_hbm)

    return k(xra, xrb, crid, cdst, cnorm, ecnt, zrows)


def _gather_rows(table, idx, d):
    bp = idx.shape[0]
    bpw = bp // (NC * NS)
    mesh = plsc.VectorSubcoreMesh(core_axis_name="c", subcore_axis_name="s")

    @functools.partial(
        pl.kernel,
        out_type=jax.ShapeDtypeStruct((bp, d), jnp.float32),
        mesh=mesh,
        compiler_params=pltpu.CompilerParams(needs_layout_passes=False),
        scratch_types=[
            pltpu.VMEM((bpw,), jnp.int32),
            pltpu.VMEM((bpw, d), jnp.float32),
            pltpu.SemaphoreType.DMA,
        ],
    )
    def k(tab_hbm, idx_hbm, out_hbm, idx_v, rows_v, sem):
        wid = lax.axis_index("s") * NC + lax.axis_index("c")
        base = wid * bpw
        pltpu.sync_copy(idx_hbm.at[pl.ds(base, bpw)], idx_v)
        pltpu.async_copy(tab_hbm.at[idx_v], rows_v, sem).wait()
        pltpu.sync_copy(rows_v, out_hbm.at[pl.ds(base, bpw)])

    return k(table, idx)


def kernel(x, edge_index, edge_type, train_node_id, w1, r1, b1, w2, r2, b2,
           fcw, fcb):
    n, _ = x.shape
    r = w1.shape[0]
    dh = w1.shape[2]
    dout = w2.shape[2]
    bsz = train_node_id.shape[0]
    half = dh // 2

    src = edge_index[0]
    dst = edge_index[1]

    wcat1 = jnp.concatenate([w1, r1[None]], axis=0)
    wcat2 = jnp.concatenate([w2, r2[None]], axis=0)
    zrows = jnp.zeros((n - 624 * (NS - 1), half), jnp.float32)

    rid, norm = _edge_prep(src, dst, edge_type, n, r)

    e = edge_type.shape[0]
    kkc = 128
    rows = -(-(-(-e // kkc)) // (2 * NS)) * (2 * NS)
    ep = rows * kkc
    rid2 = jnp.pad(rid, (0, ep - e)).reshape(rows, kkc)
    dst2 = jnp.pad(dst, (0, ep - e)).reshape(rows, kkc)
    norm2 = jnp.pad(norm, (0, ep - e)).reshape(rows, kkc)

    xa, xb = _bmm_halves(x, wcat1, bn=1000)
    agga, aggb = _msg(xa.reshape((r + 1) * n, half),
                      xb.reshape((r + 1) * n, half),
                      rid2, dst2, norm2, zrows, n, half)
    h = _relu_add(agga, aggb, xa[r], xb[r], b1.reshape(1, dh), bn=1000)

    crid, cdstc, cnorm, ecnt = _edge_compact(train_node_id, rid, dst, norm,
                                              n, cap=5120)

    ya, yb = _bmm_halves(h, wcat2, bn=1000)
    agg2a, agg2b = _msg_dyn(ya.reshape((r + 1) * n, half),
                            yb.reshape((r + 1) * n, half),
                            crid, cdstc, cnorm, ecnt, zrows, n, half, kkc)
    h2 = _relu_add(agg2a, agg2b, ya[r], yb[r], b2.reshape(1, dout), bn=1000)

    bp = 1024
    tni = jnp.pad(train_node_id, (0, bp - bsz))
    hsel = _gather_rows(h2, tni, dout)
    fcwp = jnp.pad(fcw, ((0, 0), (0, 127)))
    fcbp = jnp.pad(fcb, (0, 127)).reshape(1, 128)
    y = _fc(hsel, fcwp, fcbp)
    return y[:bsz, 0]


# speedup vs baseline: 1.0429x; 1.0429x over previous
import functools

import jax
import jax.numpy as jnp
from jax import lax
from jax.experimental import pallas as pl
from jax.experimental.pallas import tpu as pltpu
from jax.experimental.pallas import tpu_sc as plsc

NC = 2
NS = 16
LANES = 16


def _bmm_halves(a, w, bn):
    n, k = a.shape
    g, _, o = w.shape
    half = o // 2

    def body(a_ref, w_ref, oa_ref, ob_ref):
        acc = jnp.dot(a_ref[...], w_ref[0], preferred_element_type=jnp.float32)
        oa_ref[0] = acc[:, :half]
        ob_ref[0] = acc[:, half:]

    return pl.pallas_call(
        body,
        grid=(g, n // bn),
        in_specs=[
            pl.BlockSpec((bn, k), lambda gi, ni: (ni, 0)),
            pl.BlockSpec((1, k, o), lambda gi, ni: (gi, 0, 0)),
        ],
        out_specs=[
            pl.BlockSpec((1, bn, half), lambda gi, ni: (gi, ni, 0)),
            pl.BlockSpec((1, bn, half), lambda gi, ni: (gi, ni, 0)),
        ],
        out_shape=[
            jax.ShapeDtypeStruct((g, n, half), jnp.float32),
            jax.ShapeDtypeStruct((g, n, half), jnp.float32),
        ],
    )(a, w)


def _relu_add(agga, aggb, hra, hrb, bias, bn):
    n, half = agga.shape

    def body(aa, ab, ra, rb, b_ref, h_ref):
        b = b_ref[...]
        xa = jnp.maximum(aa[...] + ra[...] + b[:, :half], 0.0)
        xb = jnp.maximum(ab[...] + rb[...] + b[:, half:], 0.0)
        h_ref[...] = jnp.concatenate([xa, xb], axis=1)

    return pl.pallas_call(
        body,
        grid=(n // bn,),
        in_specs=[
            pl.BlockSpec((bn, half), lambda i: (i, 0)),
            pl.BlockSpec((bn, half), lambda i: (i, 0)),
            pl.BlockSpec((bn, half), lambda i: (i, 0)),
            pl.BlockSpec((bn, half), lambda i: (i, 0)),
            pl.BlockSpec((1, 2 * half), lambda i: (0, 0)),
        ],
        out_specs=pl.BlockSpec((bn, 2 * half), lambda i: (i, 0)),
        out_shape=jax.ShapeDtypeStruct((n, 2 * half), jnp.float32),
    )(agga, aggb, hra, hrb, bias)


def _fc(hsel, fcwp, fcbp):
    m, _ = hsel.shape

    def body(h_ref, w_ref, b_ref, o_ref):
        o_ref[...] = (
            jnp.dot(h_ref[...], w_ref[...], preferred_element_type=jnp.float32)
            + b_ref[...]
        )

    return pl.pallas_call(
        body,
        out_shape=jax.ShapeDtypeStruct((m, 128), jnp.float32),
    )(hsel, fcwp, fcbp)


def _edge_prep(src, dst, et, n_nodes, num_rel):
    e = src.shape[0]
    nr = n_nodes * num_rel
    epw = e // (NC * NS)
    ck = 4000
    cn = 1000
    mesh = plsc.VectorSubcoreMesh(core_axis_name="c", subcore_axis_name="s")

    @functools.partial(
        pl.kernel,
        out_type=[
            jax.ShapeDtypeStruct((e,), jnp.int32),
            jax.ShapeDtypeStruct((e,), jnp.float32),
        ],
        mesh=mesh,
        compiler_params=pltpu.CompilerParams(needs_layout_passes=False),
        scratch_types=[
            pltpu.VMEM((nr,), jnp.float32),
            pltpu.VMEM((ck,), jnp.int32),
            pltpu.VMEM((ck,), jnp.int32),
            pltpu.VMEM((ck,), jnp.int32),
            pltpu.VMEM((ck,), jnp.int32),
            pltpu.VMEM((cn,), jnp.int32),
            pltpu.VMEM((cn,), jnp.int32),
            pltpu.VMEM((cn,), jnp.int32),
            pltpu.VMEM((cn,), jnp.int32),
            pltpu.VMEM((cn,), jnp.float32),
            pltpu.SemaphoreType.DMA,
            pltpu.SemaphoreType.DMA,
        ],
    )
    def k(src_hbm, dst_hbm, et_hbm, rid_hbm, norm_hbm,
          counts_v, dc0, ec0, dc1, ec1, s2_v, d2_v, e2_v, ridc_v, normc_v,
          sem0, sem1):
        c = lax.axis_index("c")
        s = lax.axis_index("s")
        wid = s * NC + c
        one16 = jnp.ones((LANES,), jnp.float32)
        zero16 = jnp.zeros((LANES,), jnp.float32)

        @pl.loop(0, nr // LANES, unroll=8)
        def _(i):
            counts_v[pl.ds(i * LANES, LANES)] = zero16

        def issue(q, dcb, ecb, sem):
            base = q * ck
            pltpu.async_copy(dst_hbm.at[pl.ds(base, ck)], dcb, sem)
            pltpu.async_copy(et_hbm.at[pl.ds(base, ck)], ecb, sem)

        def waitpair(dcb, ecb, sem):
            pltpu.make_async_copy(dst_hbm.at[pl.ds(0, ck)], dcb, sem).wait()
            pltpu.make_async_copy(et_hbm.at[pl.ds(0, ck)], ecb, sem).wait()

        def count_chunk(dcb, ecb):
            @pl.loop(0, ck // LANES)
            def _(i):
                sl = pl.ds(i * LANES, LANES)
                seg = dcb[sl] * num_rel + ecb[sl]
                plsc.addupdate_scatter(counts_v, [seg], one16)

        nq = e // ck
        issue(0, dc0, ec0, sem0)

        @pl.loop(0, nq // 2)
        def _(p):
            q0 = 2 * p
            issue(q0 + 1, dc1, ec1, sem1)
            waitpair(dc0, ec0, sem0)
            count_chunk(dc0, ec0)

            @pl.when(q0 + 2 < nq)
            def _():
                issue(q0 + 2, dc0, ec0, sem0)

            waitpair(dc1, ec1, sem1)
            count_chunk(dc1, ec1)

        w0 = wid * epw

        @pl.loop(0, epw // cn)
        def _(kc):
            base = w0 + kc * cn
            pltpu.sync_copy(src_hbm.at[pl.ds(base, cn)], s2_v)
            pltpu.sync_copy(dst_hbm.at[pl.ds(base, cn)], d2_v)
            pltpu.sync_copy(et_hbm.at[pl.ds(base, cn)], e2_v)

            def group(off):
                sl = pl.ds(off, LANES)
                d16 = d2_v[sl]
                e16 = e2_v[sl]
                seg = d16 * num_rel + e16
                cnt = plsc.load_gather(counts_v, [seg])
                ridc_v[sl] = e16 * n_nodes + s2_v[sl]
                normc_v[sl] = 1.0 / jnp.maximum(cnt, 1.0)

            @pl.loop(0, cn // LANES)
            def _(i):
                group(i * LANES)

            if cn % LANES:
                group(cn - LANES)

            pltpu.sync_copy(ridc_v, rid_hbm.at[pl.ds(base, cn)])
            pltpu.sync_copy(normc_v, norm_hbm.at[pl.ds(base, cn)])

    return k(src, dst, et)


def _msg(xra, xrb, rid2, dst2, norm2, zrows, n_nodes, half):
    rows, kk = rid2.shape
    rt = rows // NS
    rst = 624
    rww = n_nodes - rst * (NS - 1)
    mesh = plsc.VectorSubcoreMesh(core_axis_name="c", subcore_axis_name="s")

    sq = 16

    @functools.partial(
        pl.kernel,
        out_type=[jax.ShapeDtypeStruct((n_nodes, half), jnp.float32)] * 2,
        mesh=mesh,
        compiler_params=pltpu.CompilerParams(needs_layout_passes=False),
        scratch_types=[
            pltpu.VMEM_SHARED((n_nodes, half), jnp.float32),
            pltpu.VMEM((sq, kk), jnp.int32),
            pltpu.VMEM((sq, kk), jnp.int32),
            pltpu.VMEM((sq, kk), jnp.float32),
            pltpu.VMEM((kk, half), jnp.float32),
            pltpu.VMEM((kk, half), jnp.float32),
            pltpu.SemaphoreType.DMA,
            pltpu.SemaphoreType.DMA,
            pltpu.SemaphoreType.DMA,
            pltpu.SemaphoreType.DMA,
        ],
    )
    def k(xra_hbm, xrb_hbm, rid_hbm, dst_hbm, norm_hbm, z_hbm,
          outa_hbm, outb_hbm, agg_sh, rid_t, dst_t, norm_t,
          msg0, msg1, gsem0, gsem1, ssem0, ssem1):
        c = lax.axis_index("c")
        s = lax.axis_index("s")

        def half_body(xr_hbm, out_hbm):
            pltpu.sync_copy(z_hbm, agg_sh.at[pl.ds(s * rst, rww)])
            plsc.subcore_barrier()

            def gather(kc, buf, sem):
                pltpu.async_copy(xr_hbm.at[rid_t.at[kc]], buf, sem)

            def wait_dma(buf, sem):
                pltpu.make_async_copy(xr_hbm.at[rid_t.at[0]], buf, sem).wait()

            def scale(kc, buf):
                @pl.loop(0, kk // LANES)
                def _(g):
                    nvec = norm_t[kc, pl.ds(g * LANES, LANES)]
                    for i in range(LANES):
                        nv = nvec[i]
                        row = g * LANES + i
                        for j in range(half // LANES):
                            sl = pl.ds(j * LANES, LANES)
                            buf[row, sl] = buf[row, sl] * nv

            def scatter(kc, buf, sem):
                pltpu.async_copy(buf, agg_sh.at[dst_t.at[kc]], sem, add=True)

            @pl.loop(0, rt // sq)
            def _(q):
                r0 = s * rt + q * sq
                pltpu.sync_copy(rid_hbm.at[pl.ds(r0, sq)], rid_t)
                pltpu.sync_copy(dst_hbm.at[pl.ds(r0, sq)], dst_t)
                pltpu.sync_copy(norm_hbm.at[pl.ds(r0, sq)], norm_t)
                gather(0, msg0, gsem0)

                @pl.loop(0, sq // 2)
                def _(p):
                    k0 = 2 * p

                    @pl.when(p > 0)
                    def _():
                        wait_dma(msg1, ssem1)

                    gather(k0 + 1, msg1, gsem1)
                    wait_dma(msg0, gsem0)
                    scale(k0, msg0)
                    scatter(k0, msg0, ssem0)
                    wait_dma(msg1, gsem1)
                    scale(k0 + 1, msg1)
                    wait_dma(msg0, ssem0)

                    @pl.when(k0 + 2 < sq)
                    def _():
                        gather(k0 + 2, msg0, gsem0)

                    scatter(k0 + 1, msg1, ssem1)

                wait_dma(msg1, ssem1)

            plsc.subcore_barrier()
            pltpu.sync_copy(agg_sh.at[pl.ds(s * rst, rww)],
                            out_hbm.at[pl.ds(s * rst, rww)])

        @pl.when(c == 0)
        def _():
            half_body(xra_hbm, outa_hbm)

        @pl.when(c == 1)
        def _():
            half_body(xrb_hbm, outb_hbm)

    return k(xra, xrb, rid2, dst2, norm2, zrows)


def _edge_compact(tni, rid, dstn, norm, n_nodes, cap):
    e = rid.shape[0]
    b = tni.shape[0]
    nw = NC * NS
    epw = e // nw
    cn = 1000
    ecap = cap * nw
    mesh = plsc.VectorSubcoreMesh(core_axis_name="c", subcore_axis_name="s")

    @functools.partial(
        pl.kernel,
        out_type=[
            jax.ShapeDtypeStruct((ecap,), jnp.int32),
            jax.ShapeDtypeStruct((ecap,), jnp.int32),
            jax.ShapeDtypeStruct((ecap,), jnp.float32),
            jax.ShapeDtypeStruct((nw * LANES,), jnp.int32),
        ],
        mesh=mesh,
        compiler_params=pltpu.CompilerParams(needs_layout_passes=False),
        scratch_types=[
            pltpu.VMEM((n_nodes,), jnp.int32),
            pltpu.VMEM((b,), jnp.int32),
            pltpu.VMEM((cn,), jnp.int32),
            pltpu.VMEM((cn,), jnp.int32),
            pltpu.VMEM((cn,), jnp.float32),
            pltpu.VMEM((cap,), jnp.int32),
            pltpu.VMEM((cap,), jnp.int32),
            pltpu.VMEM((cap,), jnp.float32),
            pltpu.VMEM((LANES,), jnp.int32),
        ],
    )
    def k(tni_hbm, rid_hbm, dst_hbm, norm_hbm,
          crid_hbm, cdst_hbm, cnorm_hbm, ecnt_hbm,
          mark_v, tni_v, ridc_v, dstc_v, nrmc_v, cr_v, cd_v, cw_v, cnt_v):
        c = lax.axis_index("c")
        s = lax.axis_index("s")
        wid = s * NC + c
        izero16 = jnp.zeros((LANES,), jnp.int32)
        ione16 = jnp.ones((LANES,), jnp.int32)
        fzero16 = jnp.zeros((LANES,), jnp.float32)
        lane = lax.iota(jnp.int32, LANES)

        @pl.loop(0, n_nodes // LANES)
        def _(i):
            mark_v[pl.ds(i * LANES, LANES)] = izero16

        pltpu.sync_copy(tni_hbm, tni_v)

        @pl.loop(0, b // LANES)
        def _(g):
            t16 = tni_v[pl.ds(g * LANES, LANES)]
            plsc.store_scatter(mark_v, [t16], ione16)

        if b % LANES:
            t16 = tni_v[pl.ds(b - LANES, LANES)]
            plsc.store_scatter(mark_v, [t16], ione16,
                               mask=lane >= LANES - (b % LANES))

        @pl.loop(0, cap // LANES)
        def _(i):
            sl = pl.ds(i * LANES, LANES)
            cr_v[sl] = izero16
            cd_v[sl] = izero16
            cw_v[sl] = fzero16

        w0 = wid * epw

        def do_group(koff, cnt, mask_extra=None):
            sl = pl.ds(koff, LANES)
            d16 = dstc_v[sl]
            r16 = ridc_v[sl]
            n16 = nrmc_v[sl]
            m16 = plsc.load_gather(mark_v, [d16]) > 0
            if mask_extra is not None:
                m16 = m16 & mask_extra
            osl = pl.ds(cnt, LANES)
            plsc.store_compressed(cr_v.at[osl], r16, mask=m16)
            plsc.store_compressed(cd_v.at[osl], d16, mask=m16)
            plsc.store_compressed(cw_v.at[osl], n16, mask=m16)
            return cnt + plsc.all_reduce_population_count(m16)[0]

        @pl.loop(0, epw // cn, init_carry=jnp.int32(0))
        def total(kc, cnt):
            base = w0 + kc * cn
            pltpu.sync_copy(rid_hbm.at[pl.ds(base, cn)], ridc_v)
            pltpu.sync_copy(dst_hbm.at[pl.ds(base, cn)], dstc_v)
            pltpu.sync_copy(norm_hbm.at[pl.ds(base, cn)], nrmc_v)

            @pl.loop(0, cn // LANES, init_carry=cnt)
            def cnt2(g, cc):
                return do_group(g * LANES, cc)

            if cn % LANES:
                cnt2 = do_group(cn - LANES, cnt2,
                                mask_extra=lane >= LANES - (cn % LANES))
            return cnt2

        pltpu.sync_copy(cr_v, crid_hbm.at[pl.ds(wid * cap, cap)])
        pltpu.sync_copy(cd_v, cdst_hbm.at[pl.ds(wid * cap, cap)])
        pltpu.sync_copy(cw_v, cnorm_hbm.at[pl.ds(wid * cap, cap)])
        cnt_v[pl.ds(0, LANES)] = jnp.full((LANES,), total, jnp.int32)
        pltpu.sync_copy(cnt_v, ecnt_hbm.at[pl.ds(wid * LANES, LANES)])

    return k(tni, rid, dstn, norm)


def _msg_dyn(xra, xrb, crid, cdst, cnorm, ecnt, zrows, n_nodes, half, kk):
    nw = NC * NS
    cap = crid.shape[0] // nw
    rst = 624
    rww = n_nodes - rst * (NS - 1)
    mesh = plsc.VectorSubcoreMesh(core_axis_name="c", subcore_axis_name="s")

    @functools.partial(
        pl.kernel,
        out_type=[jax.ShapeDtypeStruct((n_nodes, half), jnp.float32)] * 2,
        mesh=mesh,
        compiler_params=pltpu.CompilerParams(needs_layout_passes=False),
        scratch_types=[
            pltpu.VMEM_SHARED((n_nodes, half), jnp.float32),
            pltpu.VMEM((kk,), jnp.int32),
            pltpu.VMEM((kk,), jnp.int32),
            pltpu.VMEM((kk,), jnp.float32),
            pltpu.VMEM((kk, half), jnp.float32),
            pltpu.VMEM((LANES,), jnp.int32),
            pltpu.SemaphoreType.DMA,
        ],
    )
    def k(xra_hbm, xrb_hbm, crid_hbm, cdst_hbm, cnorm_hbm, ecnt_hbm, z_hbm,
          outa_hbm, outb_hbm, agg_sh, rid_v, dst_v, nrm_v, msg_v, cnt_v, sem):
        c = lax.axis_index("c")
        s = lax.axis_index("s")

        def region(xr_hbm, w):
            pltpu.sync_copy(ecnt_hbm.at[pl.ds(w * LANES, LANES)], cnt_v)
            count = cnt_v[pl.ds(0, LANES)][0]

            @pl.loop(0, cap // kk)
            def _(kc):
                @pl.when(kc * kk < count)
                def _():
                    base = w * cap + kc * kk
                    pltpu.sync_copy(crid_hbm.at[pl.ds(base, kk)], rid_v)
                    pltpu.sync_copy(cdst_hbm.at[pl.ds(base, kk)], dst_v)
                    pltpu.sync_copy(cnorm_hbm.at[pl.ds(base, kk)], nrm_v)
                    pltpu.async_copy(xr_hbm.at[rid_v], msg_v, sem).wait()

                    @pl.loop(0, kk // LANES)
                    def _(g):
                        nvec = nrm_v[pl.ds(g * LANES, LANES)]
                        for i in range(LANES):
                            nv = nvec[i]
                            row = g * LANES + i
                            for j in range(half // LANES):
                                sl = pl.ds(j * LANES, LANES)
                                msg_v[row, sl] = msg_v[row, sl] * nv

                    pltpu.sync_copy(msg_v, agg_sh.at[dst_v], add=True)

        def half_body(xr_hbm, out_hbm):
            pltpu.sync_copy(z_hbm, agg_sh.at[pl.ds(s * rst, rww)])
            plsc.subcore_barrier()
            for wofs in range(NC):
                region(xr_hbm, s * NC + wofs)
            plsc.subcore_barrier()
            pltpu.sync_copy(agg_sh.at[pl.ds(s * rst, rww)],
                            out_hbm.at[pl.ds(s * rst, rww)])

        @pl.when(c == 0)
        def _():
            half_body(xra_hbm, outa_hbm)

        @pl.when(c == 1)
        def _():
            half_body(xrb_hbm, outb_hbm)

    return k(xra, xrb, crid, cdst, cnorm, ecnt, zrows)


def _gather_rows(table, idx, d):
    bp = idx.shape[0]
    bpw = bp // (NC * NS)
    mesh = plsc.VectorSubcoreMesh(core_axis_name="c", subcore_axis_name="s")

    @functools.partial(
        pl.kernel,
        out_type=jax.ShapeDtypeStruct((bp, d), jnp.float32),
        mesh=mesh,
        compiler_params=pltpu.CompilerParams(needs_layout_passes=False),
        scratch_types=[
            pltpu.VMEM((bpw,), jnp.int32),
            pltpu.VMEM((bpw, d), jnp.float32),
            pltpu.SemaphoreType.DMA,
        ],
    )
    def k(tab_hbm, idx_hbm, out_hbm, idx_v, rows_v, sem):
        wid = lax.axis_index("s") * NC + lax.axis_index("c")
        base = wid * bpw
        pltpu.sync_copy(idx_hbm.at[pl.ds(base, bpw)], idx_v)
        pltpu.async_copy(tab_hbm.at[idx_v], rows_v, sem).wait()
        pltpu.sync_copy(rows_v, out_hbm.at[pl.ds(base, bpw)])

    return k(table, idx)


def kernel(x, edge_index, edge_type, train_node_id, w1, r1, b1, w2, r2, b2,
           fcw, fcb):
    n, _ = x.shape
    r = w1.shape[0]
    dh = w1.shape[2]
    dout = w2.shape[2]
    bsz = train_node_id.shape[0]
    half = dh // 2

    src = edge_index[0]
    dst = edge_index[1]

    wcat1 = jnp.concatenate([w1, r1[None]], axis=0)
    wcat2 = jnp.concatenate([w2, r2[None]], axis=0)
    zrows = jnp.zeros((n - 624 * (NS - 1), half), jnp.float32)

    rid, norm = _edge_prep(src, dst, edge_type, n, r)

    e = edge_type.shape[0]
    kkc = 128
    rows = -(-(-(-e // kkc)) // (2 * NS)) * (2 * NS)
    ep = rows * kkc
    rid2 = jnp.pad(rid, (0, ep - e)).reshape(rows, kkc)
    dst2 = jnp.pad(dst, (0, ep - e)).reshape(rows, kkc)
    norm2 = jnp.pad(norm, (0, ep - e)).reshape(rows, kkc)

    xa, xb = _bmm_halves(x, wcat1, bn=1000)
    agga, aggb = _msg(xa.reshape((r + 1) * n, half),
                      xb.reshape((r + 1) * n, half),
                      rid2, dst2, norm2, zrows, n, half)
    h = _relu_add(agga, aggb, xa[r], xb[r], b1.reshape(1, dh), bn=1000)

    crid, cdstc, cnorm, ecnt = _edge_compact(train_node_id, rid, dst, norm,
                                              n, cap=5120)

    ya, yb = _bmm_halves(h, wcat2, bn=1000)
    agg2a, agg2b = _msg_dyn(ya.reshape((r + 1) * n, half),
                            yb.reshape((r + 1) * n, half),
                            crid, cdstc, cnorm, ecnt, zrows, n, half, kkc)
    h2 = _relu_add(agg2a, agg2b, ya[r], yb[r], b2.reshape(1, dout), bn=1000)

    bp = 1024
    tni = jnp.pad(train_node_id, (0, bp - bsz))
    hsel = _gather_rows(h2, tni, dout)
    fcwp = jnp.pad(fcw, ((0, 0), (0, 127)))
    fcbp = jnp.pad(fcb, (0, 127)).reshape(1, 128)
    y = _fc(hsel, fcwp, fcbp)
    return y[:bsz, 0]


# speedup vs baseline: 1.1132x; 1.0675x over previous
import functools

import jax
import jax.numpy as jnp
from jax import lax
from jax.experimental import pallas as pl
from jax.experimental.pallas import tpu as pltpu
from jax.experimental.pallas import tpu_sc as plsc

NC = 2
NS = 16
LANES = 16


def _bmm_halves(a, w, bn):
    n, k = a.shape
    g, _, o = w.shape
    half = o // 2

    def body(a_ref, w_ref, oa_ref, ob_ref):
        acc = jnp.dot(a_ref[...], w_ref[0], preferred_element_type=jnp.float32)
        oa_ref[0] = acc[:, :half]
        ob_ref[0] = acc[:, half:]

    return pl.pallas_call(
        body,
        grid=(g, n // bn),
        in_specs=[
            pl.BlockSpec((bn, k), lambda gi, ni: (ni, 0)),
            pl.BlockSpec((1, k, o), lambda gi, ni: (gi, 0, 0)),
        ],
        out_specs=[
            pl.BlockSpec((1, bn, half), lambda gi, ni: (gi, ni, 0)),
            pl.BlockSpec((1, bn, half), lambda gi, ni: (gi, ni, 0)),
        ],
        out_shape=[
            jax.ShapeDtypeStruct((g, n, half), jnp.float32),
            jax.ShapeDtypeStruct((g, n, half), jnp.float32),
        ],
    )(a, w)


def _relu_add(agga, aggb, hra, hrb, bias, bn):
    n, half = agga.shape

    def body(aa, ab, ra, rb, b_ref, h_ref):
        b = b_ref[...]
        xa = jnp.maximum(aa[...] + ra[...] + b[:, :half], 0.0)
        xb = jnp.maximum(ab[...] + rb[...] + b[:, half:], 0.0)
        h_ref[...] = jnp.concatenate([xa, xb], axis=1)

    return pl.pallas_call(
        body,
        grid=(n // bn,),
        in_specs=[
            pl.BlockSpec((bn, half), lambda i: (i, 0)),
            pl.BlockSpec((bn, half), lambda i: (i, 0)),
            pl.BlockSpec((bn, half), lambda i: (i, 0)),
            pl.BlockSpec((bn, half), lambda i: (i, 0)),
            pl.BlockSpec((1, 2 * half), lambda i: (0, 0)),
        ],
        out_specs=pl.BlockSpec((bn, 2 * half), lambda i: (i, 0)),
        out_shape=jax.ShapeDtypeStruct((n, 2 * half), jnp.float32),
    )(agga, aggb, hra, hrb, bias)


def _fc(hsel, fcwp, fcbp):
    m, _ = hsel.shape

    def body(h_ref, w_ref, b_ref, o_ref):
        o_ref[...] = (
            jnp.dot(h_ref[...], w_ref[...], preferred_element_type=jnp.float32)
            + b_ref[...]
        )

    return pl.pallas_call(
        body,
        out_shape=jax.ShapeDtypeStruct((m, 128), jnp.float32),
    )(hsel, fcwp, fcbp)


def _edge_prep(src, dst, et, n_nodes, num_rel):
    e = src.shape[0]
    nr = n_nodes * num_rel
    epw = e // (NC * NS)
    ck = 4000
    cn = 1000
    mesh = plsc.VectorSubcoreMesh(core_axis_name="c", subcore_axis_name="s")

    @functools.partial(
        pl.kernel,
        out_type=[
            jax.ShapeDtypeStruct((e,), jnp.int32),
            jax.ShapeDtypeStruct((e,), jnp.float32),
        ],
        mesh=mesh,
        compiler_params=pltpu.CompilerParams(needs_layout_passes=False),
        scratch_types=[
            pltpu.VMEM((nr,), jnp.float32),
            pltpu.VMEM((ck,), jnp.int32),
            pltpu.VMEM((ck,), jnp.int32),
            pltpu.VMEM((ck,), jnp.int32),
            pltpu.VMEM((ck,), jnp.int32),
            pltpu.VMEM((cn,), jnp.int32),
            pltpu.VMEM((cn,), jnp.int32),
            pltpu.VMEM((cn,), jnp.int32),
            pltpu.VMEM((cn,), jnp.int32),
            pltpu.VMEM((cn,), jnp.float32),
            pltpu.SemaphoreType.DMA,
            pltpu.SemaphoreType.DMA,
        ],
    )
    def k(src_hbm, dst_hbm, et_hbm, rid_hbm, norm_hbm,
          counts_v, dc0, ec0, dc1, ec1, s2_v, d2_v, e2_v, ridc_v, normc_v,
          sem0, sem1):
        c = lax.axis_index("c")
        s = lax.axis_index("s")
        wid = s * NC + c
        one16 = jnp.ones((LANES,), jnp.float32)
        zero16 = jnp.zeros((LANES,), jnp.float32)

        @pl.loop(0, nr // LANES, unroll=8)
        def _(i):
            counts_v[pl.ds(i * LANES, LANES)] = zero16

        def issue(q, dcb, ecb, sem):
            base = q * ck
            pltpu.async_copy(dst_hbm.at[pl.ds(base, ck)], dcb, sem)
            pltpu.async_copy(et_hbm.at[pl.ds(base, ck)], ecb, sem)

        def waitpair(dcb, ecb, sem):
            pltpu.make_async_copy(dst_hbm.at[pl.ds(0, ck)], dcb, sem).wait()
            pltpu.make_async_copy(et_hbm.at[pl.ds(0, ck)], ecb, sem).wait()

        def count_chunk(dcb, ecb):
            @pl.loop(0, ck // LANES)
            def _(i):
                sl = pl.ds(i * LANES, LANES)
                seg = dcb[sl] * num_rel + ecb[sl]
                plsc.addupdate_scatter(counts_v, [seg], one16)

        nq = e // ck
        issue(0, dc0, ec0, sem0)

        @pl.loop(0, nq // 2)
        def _(p):
            q0 = 2 * p
            issue(q0 + 1, dc1, ec1, sem1)
            waitpair(dc0, ec0, sem0)
            count_chunk(dc0, ec0)

            @pl.when(q0 + 2 < nq)
            def _():
                issue(q0 + 2, dc0, ec0, sem0)

            waitpair(dc1, ec1, sem1)
            count_chunk(dc1, ec1)

        w0 = wid * epw

        @pl.loop(0, epw // cn)
        def _(kc):
            base = w0 + kc * cn
            pltpu.sync_copy(src_hbm.at[pl.ds(base, cn)], s2_v)
            pltpu.sync_copy(dst_hbm.at[pl.ds(base, cn)], d2_v)
            pltpu.sync_copy(et_hbm.at[pl.ds(base, cn)], e2_v)

            def group(off):
                sl = pl.ds(off, LANES)
                d16 = d2_v[sl]
                e16 = e2_v[sl]
                seg = d16 * num_rel + e16
                cnt = plsc.load_gather(counts_v, [seg])
                ridc_v[sl] = e16 * n_nodes + s2_v[sl]
                normc_v[sl] = 1.0 / jnp.maximum(cnt, 1.0)

            @pl.loop(0, cn // LANES)
            def _(i):
                group(i * LANES)

            if cn % LANES:
                group(cn - LANES)

            pltpu.sync_copy(ridc_v, rid_hbm.at[pl.ds(base, cn)])
            pltpu.sync_copy(normc_v, norm_hbm.at[pl.ds(base, cn)])

    return k(src, dst, et)


def _msg(xra, xrb, rid2, dst2, norm2, zrows, n_nodes, half):
    rows, kk = rid2.shape
    rt = rows // NS
    rst = 624
    rww = n_nodes - rst * (NS - 1)
    mesh = plsc.VectorSubcoreMesh(core_axis_name="c", subcore_axis_name="s")

    sq = 16

    @functools.partial(
        pl.kernel,
        out_type=[jax.ShapeDtypeStruct((n_nodes, half), jnp.float32)] * 2,
        mesh=mesh,
        compiler_params=pltpu.CompilerParams(needs_layout_passes=False),
        scratch_types=[
            pltpu.VMEM_SHARED((n_nodes, half), jnp.float32),
            pltpu.VMEM((sq, kk), jnp.int32),
            pltpu.VMEM((sq, kk), jnp.int32),
            pltpu.VMEM((sq, kk), jnp.float32),
            pltpu.VMEM((kk, half), jnp.float32),
            pltpu.VMEM((kk, half), jnp.float32),
            pltpu.SemaphoreType.DMA,
            pltpu.SemaphoreType.DMA,
            pltpu.SemaphoreType.DMA,
            pltpu.SemaphoreType.DMA,
        ],
    )
    def k(xra_hbm, xrb_hbm, rid_hbm, dst_hbm, norm_hbm, z_hbm,
          outa_hbm, outb_hbm, agg_sh, rid_t, dst_t, norm_t,
          msg0, msg1, gsem0, gsem1, ssem0, ssem1):
        c = lax.axis_index("c")
        s = lax.axis_index("s")

        def half_body(xr_hbm, out_hbm):
            pltpu.sync_copy(z_hbm, agg_sh.at[pl.ds(s * rst, rww)])
            plsc.subcore_barrier()

            def gather(kc, buf, sem):
                pltpu.async_copy(xr_hbm.at[rid_t.at[kc]], buf, sem)

            def wait_dma(buf, sem):
                pltpu.make_async_copy(xr_hbm.at[rid_t.at[0]], buf, sem).wait()

            def scale(kc, buf):
                @pl.loop(0, kk // LANES)
                def _(g):
                    nvec = norm_t[kc, pl.ds(g * LANES, LANES)]
                    for i in range(LANES):
                        nv = nvec[i]
                        row = g * LANES + i
                        for j in range(half // LANES):
                            sl = pl.ds(j * LANES, LANES)
                            buf[row, sl] = buf[row, sl] * nv

            def scatter(kc, buf, sem):
                pltpu.async_copy(buf, agg_sh.at[dst_t.at[kc]], sem, add=True)

            @pl.loop(0, rt // sq)
            def _(q):
                r0 = s * rt + q * sq
                pltpu.sync_copy(rid_hbm.at[pl.ds(r0, sq)], rid_t)
                pltpu.sync_copy(dst_hbm.at[pl.ds(r0, sq)], dst_t)
                pltpu.sync_copy(norm_hbm.at[pl.ds(r0, sq)], norm_t)
                gather(0, msg0, gsem0)

                @pl.loop(0, sq // 2)
                def _(p):
                    k0 = 2 * p

                    @pl.when(p > 0)
                    def _():
                        wait_dma(msg1, ssem1)

                    gather(k0 + 1, msg1, gsem1)
                    wait_dma(msg0, gsem0)
                    scale(k0, msg0)
                    scatter(k0, msg0, ssem0)
                    wait_dma(msg1, gsem1)
                    scale(k0 + 1, msg1)
                    wait_dma(msg0, ssem0)

                    @pl.when(k0 + 2 < sq)
                    def _():
                        gather(k0 + 2, msg0, gsem0)

                    scatter(k0 + 1, msg1, ssem1)

                wait_dma(msg1, ssem1)

            plsc.subcore_barrier()
            pltpu.sync_copy(agg_sh.at[pl.ds(s * rst, rww)],
                            out_hbm.at[pl.ds(s * rst, rww)])

        @pl.when(c == 0)
        def _():
            half_body(xra_hbm, outa_hbm)

        @pl.when(c == 1)
        def _():
            half_body(xrb_hbm, outb_hbm)

    return k(xra, xrb, rid2, dst2, norm2, zrows)


def _edge_compact(tni, rid, dstn, norm, n_nodes, cap):
    e = rid.shape[0]
    b = tni.shape[0]
    nw = NC * NS
    epw = e // nw
    cn = 1000
    ecap = cap * nw
    mesh = plsc.VectorSubcoreMesh(core_axis_name="c", subcore_axis_name="s")

    @functools.partial(
        pl.kernel,
        out_type=[
            jax.ShapeDtypeStruct((ecap,), jnp.int32),
            jax.ShapeDtypeStruct((ecap,), jnp.int32),
            jax.ShapeDtypeStruct((ecap,), jnp.float32),
            jax.ShapeDtypeStruct((nw * LANES,), jnp.int32),
        ],
        mesh=mesh,
        compiler_params=pltpu.CompilerParams(needs_layout_passes=False),
        scratch_types=[
            pltpu.VMEM((n_nodes,), jnp.int32),
            pltpu.VMEM((b,), jnp.int32),
            pltpu.VMEM((cn,), jnp.int32),
            pltpu.VMEM((cn,), jnp.int32),
            pltpu.VMEM((cn,), jnp.float32),
            pltpu.VMEM((cap,), jnp.int32),
            pltpu.VMEM((cap,), jnp.int32),
            pltpu.VMEM((cap,), jnp.float32),
            pltpu.VMEM((LANES,), jnp.int32),
        ],
    )
    def k(tni_hbm, rid_hbm, dst_hbm, norm_hbm,
          crid_hbm, cdst_hbm, cnorm_hbm, ecnt_hbm,
          mark_v, tni_v, ridc_v, dstc_v, nrmc_v, cr_v, cd_v, cw_v, cnt_v):
        c = lax.axis_index("c")
        s = lax.axis_index("s")
        wid = s * NC + c
        izero16 = jnp.zeros((LANES,), jnp.int32)
        ione16 = jnp.ones((LANES,), jnp.int32)
        fzero16 = jnp.zeros((LANES,), jnp.float32)
        lane = lax.iota(jnp.int32, LANES)

        @pl.loop(0, n_nodes // LANES)
        def _(i):
            mark_v[pl.ds(i * LANES, LANES)] = izero16

        pltpu.sync_copy(tni_hbm, tni_v)

        @pl.loop(0, b // LANES)
        def _(g):
            t16 = tni_v[pl.ds(g * LANES, LANES)]
            plsc.store_scatter(mark_v, [t16], ione16)

        if b % LANES:
            t16 = tni_v[pl.ds(b - LANES, LANES)]
            plsc.store_scatter(mark_v, [t16], ione16,
                               mask=lane >= LANES - (b % LANES))

        @pl.loop(0, cap // LANES)
        def _(i):
            sl = pl.ds(i * LANES, LANES)
            cr_v[sl] = izero16
            cd_v[sl] = izero16
            cw_v[sl] = fzero16

        w0 = wid * epw

        def do_group(koff, cnt, mask_extra=None):
            sl = pl.ds(koff, LANES)
            d16 = dstc_v[sl]
            r16 = ridc_v[sl]
            n16 = nrmc_v[sl]
            m16 = plsc.load_gather(mark_v, [d16]) > 0
            if mask_extra is not None:
                m16 = m16 & mask_extra
            osl = pl.ds(cnt, LANES)
            plsc.store_compressed(cr_v.at[osl], r16, mask=m16)
            plsc.store_compressed(cd_v.at[osl], d16, mask=m16)
            plsc.store_compressed(cw_v.at[osl], n16, mask=m16)
            return cnt + plsc.all_reduce_population_count(m16)[0]

        @pl.loop(0, epw // cn, init_carry=jnp.int32(0))
        def total(kc, cnt):
            base = w0 + kc * cn
            pltpu.sync_copy(rid_hbm.at[pl.ds(base, cn)], ridc_v)
            pltpu.sync_copy(dst_hbm.at[pl.ds(base, cn)], dstc_v)
            pltpu.sync_copy(norm_hbm.at[pl.ds(base, cn)], nrmc_v)

            @pl.loop(0, cn // LANES, init_carry=cnt)
            def cnt2(g, cc):
                return do_group(g * LANES, cc)

            if cn % LANES:
                cnt2 = do_group(cn - LANES, cnt2,
                                mask_extra=lane >= LANES - (cn % LANES))
            return cnt2

        pltpu.sync_copy(cr_v, crid_hbm.at[pl.ds(wid * cap, cap)])
        pltpu.sync_copy(cd_v, cdst_hbm.at[pl.ds(wid * cap, cap)])
        pltpu.sync_copy(cw_v, cnorm_hbm.at[pl.ds(wid * cap, cap)])
        cnt_v[pl.ds(0, LANES)] = jnp.full((LANES,), total, jnp.int32)
        pltpu.sync_copy(cnt_v, ecnt_hbm.at[pl.ds(wid * LANES, LANES)])

    return k(tni, rid, dstn, norm)


def _msg_dyn(xra, xrb, crid, cdst, cnorm, ecnt, zrows, n_nodes, half, kk):
    nw = NC * NS
    cap = crid.shape[0] // nw
    rst = 624
    rww = n_nodes - rst * (NS - 1)
    mesh = plsc.VectorSubcoreMesh(core_axis_name="c", subcore_axis_name="s")

    @functools.partial(
        pl.kernel,
        out_type=[jax.ShapeDtypeStruct((n_nodes, half), jnp.float32)] * 2,
        mesh=mesh,
        compiler_params=pltpu.CompilerParams(needs_layout_passes=False),
        scratch_types=[
            pltpu.VMEM_SHARED((n_nodes, half), jnp.float32),
            pltpu.VMEM((kk,), jnp.int32),
            pltpu.VMEM((kk,), jnp.int32),
            pltpu.VMEM((kk,), jnp.float32),
            pltpu.VMEM((kk, half), jnp.float32),
            pltpu.VMEM((LANES,), jnp.int32),
            pltpu.SemaphoreType.DMA,
        ],
    )
    def k(xra_hbm, xrb_hbm, crid_hbm, cdst_hbm, cnorm_hbm, ecnt_hbm, z_hbm,
          outa_hbm, outb_hbm, agg_sh, rid_v, dst_v, nrm_v, msg_v, cnt_v, sem):
        c = lax.axis_index("c")
        s = lax.axis_index("s")

        def region(xr_hbm, w):
            pltpu.sync_copy(ecnt_hbm.at[pl.ds(w * LANES, LANES)], cnt_v)
            count = cnt_v[pl.ds(0, LANES)][0]

            @pl.loop(0, cap // kk)
            def _(kc):
                @pl.when(kc * kk < count)
                def _():
                    base = w * cap + kc * kk
                    pltpu.sync_copy(crid_hbm.at[pl.ds(base, kk)], rid_v)
                    pltpu.sync_copy(cdst_hbm.at[pl.ds(base, kk)], dst_v)
                    pltpu.sync_copy(cnorm_hbm.at[pl.ds(base, kk)], nrm_v)
                    pltpu.async_copy(xr_hbm.at[rid_v], msg_v, sem).wait()

                    @pl.loop(0, kk // LANES)
                    def _(g):
                        nvec = nrm_v[pl.ds(g * LANES, LANES)]
                        for i in range(LANES):
                            nv = nvec[i]
                            row = g * LANES + i
                            for j in range(half // LANES):
                                sl = pl.ds(j * LANES, LANES)
                                msg_v[row, sl] = msg_v[row, sl] * nv

                    pltpu.sync_copy(msg_v, agg_sh.at[dst_v], add=True)

        def half_body(xr_hbm, out_hbm):
            pltpu.sync_copy(z_hbm, agg_sh.at[pl.ds(s * rst, rww)])
            plsc.subcore_barrier()
            for wofs in range(NC):
                region(xr_hbm, s * NC + wofs)
            plsc.subcore_barrier()
            pltpu.sync_copy(agg_sh.at[pl.ds(s * rst, rww)],
                            out_hbm.at[pl.ds(s * rst, rww)])

        @pl.when(c == 0)
        def _():
            half_body(xra_hbm, outa_hbm)

        @pl.when(c == 1)
        def _():
            half_body(xrb_hbm, outb_hbm)

    return k(xra, xrb, crid, cdst, cnorm, ecnt, zrows)


def _gather_rows(table, idx, d):
    bp = idx.shape[0]
    bpw = bp // (NC * NS)
    mesh = plsc.VectorSubcoreMesh(core_axis_name="c", subcore_axis_name="s")

    @functools.partial(
        pl.kernel,
        out_type=jax.ShapeDtypeStruct((bp, d), jnp.float32),
        mesh=mesh,
        compiler_params=pltpu.CompilerParams(needs_layout_passes=False),
        scratch_types=[
            pltpu.VMEM((bpw,), jnp.int32),
            pltpu.VMEM((bpw, d), jnp.float32),
            pltpu.SemaphoreType.DMA,
        ],
    )
    def k(tab_hbm, idx_hbm, out_hbm, idx_v, rows_v, sem):
        wid = lax.axis_index("s") * NC + lax.axis_index("c")
        base = wid * bpw
        pltpu.sync_copy(idx_hbm.at[pl.ds(base, bpw)], idx_v)
        pltpu.async_copy(tab_hbm.at[idx_v], rows_v, sem).wait()
        pltpu.sync_copy(rows_v, out_hbm.at[pl.ds(base, bpw)])

    return k(table, idx)


def kernel(x, edge_index, edge_type, train_node_id, w1, r1, b1, w2, r2, b2,
           fcw, fcb):
    n, _ = x.shape
    r = w1.shape[0]
    dh = w1.shape[2]
    dout = w2.shape[2]
    bsz = train_node_id.shape[0]
    half = dh // 2

    src = edge_index[0]
    dst = edge_index[1]

    wcat1 = jnp.concatenate([w1, r1[None]], axis=0)
    wcat2 = jnp.concatenate([w2, r2[None]], axis=0)
    zrows = jnp.zeros((n - 624 * (NS - 1), half), jnp.float32)

    rid, norm = _edge_prep(src, dst, edge_type, n, r)

    e = edge_type.shape[0]
    kkc = 128
    rows = -(-(-(-e // kkc)) // (2 * NS)) * (2 * NS)
    ep = rows * kkc
    rid2 = jnp.pad(rid, (0, ep - e)).reshape(rows, kkc)
    dst2 = jnp.pad(dst, (0, ep - e)).reshape(rows, kkc)
    norm2 = jnp.pad(norm, (0, ep - e)).reshape(rows, kkc)

    wcat1b = wcat1.astype(jnp.bfloat16)
    wcat2b = wcat2.astype(jnp.bfloat16)
    xa, xb = _bmm_halves(x.astype(jnp.bfloat16), wcat1b, bn=1000)
    agga, aggb = _msg(xa.reshape((r + 1) * n, half),
                      xb.reshape((r + 1) * n, half),
                      rid2, dst2, norm2, zrows, n, half)
    h = _relu_add(agga, aggb, xa[r], xb[r], b1.reshape(1, dh), bn=1000)

    crid, cdstc, cnorm, ecnt = _edge_compact(train_node_id, rid, dst, norm,
                                              n, cap=5120)

    ya, yb = _bmm_halves(h.astype(jnp.bfloat16), wcat2b, bn=1000)
    agg2a, agg2b = _msg_dyn(ya.reshape((r + 1) * n, half),
                            yb.reshape((r + 1) * n, half),
                            crid, cdstc, cnorm, ecnt, zrows, n, half, kkc)
    h2 = _relu_add(agg2a, agg2b, ya[r], yb[r], b2.reshape(1, dout), bn=1000)

    bp = 1024
    tni = jnp.pad(train_node_id, (0, bp - bsz))
    hsel = _gather_rows(h2, tni, dout)
    fcwp = jnp.pad(fcw, ((0, 0), (0, 127)))
    fcbp = jnp.pad(fcb, (0, 127)).reshape(1, 128)
    y = _fc(hsel, fcwp, fcbp)
    return y[:bsz, 0]


# speedup vs baseline: 1.2217x; 1.0974x over previous
import functools

import jax
import jax.numpy as jnp
from jax import lax
from jax.experimental import pallas as pl
from jax.experimental.pallas import tpu as pltpu
from jax.experimental.pallas import tpu_sc as plsc

NC = 2
NS = 16
LANES = 16


def _bmm_halves(a, w, bn):
    n, k = a.shape
    g, _, o = w.shape
    half = o // 2

    def body(a_ref, w_ref, oa_ref, ob_ref):
        a_blk = a_ref[...]
        for gi in range(g):
            acc = jnp.dot(a_blk, w_ref[gi], preferred_element_type=jnp.float32)
            oa_ref[gi] = acc[:, :half]
            ob_ref[gi] = acc[:, half:]

    return pl.pallas_call(
        body,
        grid=(n // bn,),
        in_specs=[
            pl.BlockSpec((bn, k), lambda ni: (ni, 0)),
            pl.BlockSpec((g, k, o), lambda ni: (0, 0, 0)),
        ],
        out_specs=[
            pl.BlockSpec((g, bn, half), lambda ni: (0, ni, 0)),
            pl.BlockSpec((g, bn, half), lambda ni: (0, ni, 0)),
        ],
        out_shape=[
            jax.ShapeDtypeStruct((g, n, half), jnp.float32),
            jax.ShapeDtypeStruct((g, n, half), jnp.float32),
        ],
    )(a, w)


def _relu_add(agga, aggb, hra, hrb, bias, bn):
    n, half = agga.shape

    def body(aa, ab, ra, rb, b_ref, h_ref):
        b = b_ref[...]
        xa = jnp.maximum(aa[...] + ra[...] + b[:, :half], 0.0)
        xb = jnp.maximum(ab[...] + rb[...] + b[:, half:], 0.0)
        h_ref[...] = jnp.concatenate([xa, xb], axis=1)

    return pl.pallas_call(
        body,
        grid=(n // bn,),
        in_specs=[
            pl.BlockSpec((bn, half), lambda i: (i, 0)),
            pl.BlockSpec((bn, half), lambda i: (i, 0)),
            pl.BlockSpec((bn, half), lambda i: (i, 0)),
            pl.BlockSpec((bn, half), lambda i: (i, 0)),
            pl.BlockSpec((1, 2 * half), lambda i: (0, 0)),
        ],
        out_specs=pl.BlockSpec((bn, 2 * half), lambda i: (i, 0)),
        out_shape=jax.ShapeDtypeStruct((n, 2 * half), jnp.float32),
    )(agga, aggb, hra, hrb, bias)


def _fc(hsel, fcwp, fcbp):
    m, _ = hsel.shape

    def body(h_ref, w_ref, b_ref, o_ref):
        o_ref[...] = (
            jnp.dot(h_ref[...], w_ref[...], preferred_element_type=jnp.float32)
            + b_ref[...]
        )

    return pl.pallas_call(
        body,
        out_shape=jax.ShapeDtypeStruct((m, 128), jnp.float32),
    )(hsel, fcwp, fcbp)


def _edge_prep(src, dst, et, n_nodes, num_rel):
    e = src.shape[0]
    nr = n_nodes * num_rel
    epw = e // (NC * NS)
    ck = 4000
    cn = 1000
    mesh = plsc.VectorSubcoreMesh(core_axis_name="c", subcore_axis_name="s")

    @functools.partial(
        pl.kernel,
        out_type=[
            jax.ShapeDtypeStruct((e,), jnp.int32),
            jax.ShapeDtypeStruct((e,), jnp.float32),
        ],
        mesh=mesh,
        compiler_params=pltpu.CompilerParams(needs_layout_passes=False),
        scratch_types=[
            pltpu.VMEM((nr,), jnp.float32),
            pltpu.VMEM((ck,), jnp.int32),
            pltpu.VMEM((ck,), jnp.int32),
            pltpu.VMEM((ck,), jnp.int32),
            pltpu.VMEM((ck,), jnp.int32),
            pltpu.VMEM((cn,), jnp.int32),
            pltpu.VMEM((cn,), jnp.int32),
            pltpu.VMEM((cn,), jnp.int32),
            pltpu.VMEM((cn,), jnp.int32),
            pltpu.VMEM((cn,), jnp.float32),
            pltpu.SemaphoreType.DMA,
            pltpu.SemaphoreType.DMA,
        ],
    )
    def k(src_hbm, dst_hbm, et_hbm, rid_hbm, norm_hbm,
          counts_v, dc0, ec0, dc1, ec1, s2_v, d2_v, e2_v, ridc_v, normc_v,
          sem0, sem1):
        c = lax.axis_index("c")
        s = lax.axis_index("s")
        wid = s * NC + c
        one16 = jnp.ones((LANES,), jnp.float32)
        zero16 = jnp.zeros((LANES,), jnp.float32)

        @pl.loop(0, nr // LANES, unroll=8)
        def _(i):
            counts_v[pl.ds(i * LANES, LANES)] = zero16

        def issue(q, dcb, ecb, sem):
            base = q * ck
            pltpu.async_copy(dst_hbm.at[pl.ds(base, ck)], dcb, sem)
            pltpu.async_copy(et_hbm.at[pl.ds(base, ck)], ecb, sem)

        def waitpair(dcb, ecb, sem):
            pltpu.make_async_copy(dst_hbm.at[pl.ds(0, ck)], dcb, sem).wait()
            pltpu.make_async_copy(et_hbm.at[pl.ds(0, ck)], ecb, sem).wait()

        def count_chunk(dcb, ecb):
            @pl.loop(0, ck // LANES)
            def _(i):
                sl = pl.ds(i * LANES, LANES)
                seg = dcb[sl] * num_rel + ecb[sl]
                plsc.addupdate_scatter(counts_v, [seg], one16)

        nq = e // ck
        issue(0, dc0, ec0, sem0)

        @pl.loop(0, nq // 2)
        def _(p):
            q0 = 2 * p
            issue(q0 + 1, dc1, ec1, sem1)
            waitpair(dc0, ec0, sem0)
            count_chunk(dc0, ec0)

            @pl.when(q0 + 2 < nq)
            def _():
                issue(q0 + 2, dc0, ec0, sem0)

            waitpair(dc1, ec1, sem1)
            count_chunk(dc1, ec1)

        w0 = wid * epw

        @pl.loop(0, epw // cn)
        def _(kc):
            base = w0 + kc * cn
            pltpu.sync_copy(src_hbm.at[pl.ds(base, cn)], s2_v)
            pltpu.sync_copy(dst_hbm.at[pl.ds(base, cn)], d2_v)
            pltpu.sync_copy(et_hbm.at[pl.ds(base, cn)], e2_v)

            def group(off):
                sl = pl.ds(off, LANES)
                d16 = d2_v[sl]
                e16 = e2_v[sl]
                seg = d16 * num_rel + e16
                cnt = plsc.load_gather(counts_v, [seg])
                ridc_v[sl] = e16 * n_nodes + s2_v[sl]
                normc_v[sl] = 1.0 / jnp.maximum(cnt, 1.0)

            @pl.loop(0, cn // LANES)
            def _(i):
                group(i * LANES)

            if cn % LANES:
                group(cn - LANES)

            pltpu.sync_copy(ridc_v, rid_hbm.at[pl.ds(base, cn)])
            pltpu.sync_copy(normc_v, norm_hbm.at[pl.ds(base, cn)])

    return k(src, dst, et)


def _msg(xra, xrb, rid2, dst2, norm2, zrows, n_nodes, half):
    rows, kk = rid2.shape
    rt = rows // NS
    rst = 624
    rww = n_nodes - rst * (NS - 1)
    mesh = plsc.VectorSubcoreMesh(core_axis_name="c", subcore_axis_name="s")

    sq = 16

    @functools.partial(
        pl.kernel,
        out_type=[jax.ShapeDtypeStruct((n_nodes, half), jnp.float32)] * 2,
        mesh=mesh,
        compiler_params=pltpu.CompilerParams(needs_layout_passes=False),
        scratch_types=[
            pltpu.VMEM_SHARED((n_nodes, half), jnp.float32),
            pltpu.VMEM((sq, kk), jnp.int32),
            pltpu.VMEM((sq, kk), jnp.int32),
            pltpu.VMEM((sq, kk), jnp.float32),
            pltpu.VMEM((kk, half), jnp.float32),
            pltpu.VMEM((kk, half), jnp.float32),
            pltpu.SemaphoreType.DMA,
            pltpu.SemaphoreType.DMA,
            pltpu.SemaphoreType.DMA,
            pltpu.SemaphoreType.DMA,
        ],
    )
    def k(xra_hbm, xrb_hbm, rid_hbm, dst_hbm, norm_hbm, z_hbm,
          outa_hbm, outb_hbm, agg_sh, rid_t, dst_t, norm_t,
          msg0, msg1, gsem0, gsem1, ssem0, ssem1):
        c = lax.axis_index("c")
        s = lax.axis_index("s")

        def half_body(xr_hbm, out_hbm):
            pltpu.sync_copy(z_hbm, agg_sh.at[pl.ds(s * rst, rww)])
            plsc.subcore_barrier()

            def gather(kc, buf, sem):
                pltpu.async_copy(xr_hbm.at[rid_t.at[kc]], buf, sem)

            def wait_dma(buf, sem):
                pltpu.make_async_copy(xr_hbm.at[rid_t.at[0]], buf, sem).wait()

            def scale(kc, buf):
                @pl.loop(0, kk // LANES)
                def _(g):
                    nvec = norm_t[kc, pl.ds(g * LANES, LANES)]
                    for i in range(LANES):
                        nv = nvec[i]
                        row = g * LANES + i
                        for j in range(half // LANES):
                            sl = pl.ds(j * LANES, LANES)
                            buf[row, sl] = buf[row, sl] * nv

            def scatter(kc, buf, sem):
                pltpu.async_copy(buf, agg_sh.at[dst_t.at[kc]], sem, add=True)

            @pl.loop(0, rt // sq)
            def _(q):
                r0 = s * rt + q * sq
                pltpu.sync_copy(rid_hbm.at[pl.ds(r0, sq)], rid_t)
                pltpu.sync_copy(dst_hbm.at[pl.ds(r0, sq)], dst_t)
                pltpu.sync_copy(norm_hbm.at[pl.ds(r0, sq)], norm_t)
                gather(0, msg0, gsem0)

                @pl.loop(0, sq // 2)
                def _(p):
                    k0 = 2 * p

                    @pl.when(p > 0)
                    def _():
                        wait_dma(msg1, ssem1)

                    gather(k0 + 1, msg1, gsem1)
                    wait_dma(msg0, gsem0)
                    scale(k0, msg0)
                    scatter(k0, msg0, ssem0)
                    wait_dma(msg1, gsem1)
                    scale(k0 + 1, msg1)
                    wait_dma(msg0, ssem0)

                    @pl.when(k0 + 2 < sq)
                    def _():
                        gather(k0 + 2, msg0, gsem0)

                    scatter(k0 + 1, msg1, ssem1)

                wait_dma(msg1, ssem1)

            plsc.subcore_barrier()
            pltpu.sync_copy(agg_sh.at[pl.ds(s * rst, rww)],
                            out_hbm.at[pl.ds(s * rst, rww)])

        @pl.when(c == 0)
        def _():
            half_body(xra_hbm, outa_hbm)

        @pl.when(c == 1)
        def _():
            half_body(xrb_hbm, outb_hbm)

    return k(xra, xrb, rid2, dst2, norm2, zrows)


def _edge_compact(tni, rid, dstn, norm, n_nodes, cap):
    e = rid.shape[0]
    b = tni.shape[0]
    nw = NC * NS
    epw = e // nw
    cn = 1000
    ecap = cap * nw
    mesh = plsc.VectorSubcoreMesh(core_axis_name="c", subcore_axis_name="s")

    @functools.partial(
        pl.kernel,
        out_type=[
            jax.ShapeDtypeStruct((ecap,), jnp.int32),
            jax.ShapeDtypeStruct((ecap,), jnp.int32),
            jax.ShapeDtypeStruct((ecap,), jnp.float32),
            jax.ShapeDtypeStruct((nw * LANES,), jnp.int32),
        ],
        mesh=mesh,
        compiler_params=pltpu.CompilerParams(needs_layout_passes=False),
        scratch_types=[
            pltpu.VMEM((n_nodes,), jnp.int32),
            pltpu.VMEM((b,), jnp.int32),
            pltpu.VMEM((cn,), jnp.int32),
            pltpu.VMEM((cn,), jnp.int32),
            pltpu.VMEM((cn,), jnp.float32),
            pltpu.VMEM((cap,), jnp.int32),
            pltpu.VMEM((cap,), jnp.int32),
            pltpu.VMEM((cap,), jnp.float32),
            pltpu.VMEM((LANES,), jnp.int32),
        ],
    )
    def k(tni_hbm, rid_hbm, dst_hbm, norm_hbm,
          crid_hbm, cdst_hbm, cnorm_hbm, ecnt_hbm,
          mark_v, tni_v, ridc_v, dstc_v, nrmc_v, cr_v, cd_v, cw_v, cnt_v):
        c = lax.axis_index("c")
        s = lax.axis_index("s")
        wid = s * NC + c
        izero16 = jnp.zeros((LANES,), jnp.int32)
        ione16 = jnp.ones((LANES,), jnp.int32)
        fzero16 = jnp.zeros((LANES,), jnp.float32)
        lane = lax.iota(jnp.int32, LANES)

        @pl.loop(0, n_nodes // LANES)
        def _(i):
            mark_v[pl.ds(i * LANES, LANES)] = izero16

        pltpu.sync_copy(tni_hbm, tni_v)

        @pl.loop(0, b // LANES)
        def _(g):
            t16 = tni_v[pl.ds(g * LANES, LANES)]
            plsc.store_scatter(mark_v, [t16], ione16)

        if b % LANES:
            t16 = tni_v[pl.ds(b - LANES, LANES)]
            plsc.store_scatter(mark_v, [t16], ione16,
                               mask=lane >= LANES - (b % LANES))

        @pl.loop(0, cap // LANES)
        def _(i):
            sl = pl.ds(i * LANES, LANES)
            cr_v[sl] = izero16
            cd_v[sl] = izero16
            cw_v[sl] = fzero16

        w0 = wid * epw

        def do_group(koff, cnt, mask_extra=None):
            sl = pl.ds(koff, LANES)
            d16 = dstc_v[sl]
            r16 = ridc_v[sl]
            n16 = nrmc_v[sl]
            m16 = plsc.load_gather(mark_v, [d16]) > 0
            if mask_extra is not None:
                m16 = m16 & mask_extra
            osl = pl.ds(cnt, LANES)
            plsc.store_compressed(cr_v.at[osl], r16, mask=m16)
            plsc.store_compressed(cd_v.at[osl], d16, mask=m16)
            plsc.store_compressed(cw_v.at[osl], n16, mask=m16)
            return cnt + plsc.all_reduce_population_count(m16)[0]

        @pl.loop(0, epw // cn, init_carry=jnp.int32(0))
        def total(kc, cnt):
            base = w0 + kc * cn
            pltpu.sync_copy(rid_hbm.at[pl.ds(base, cn)], ridc_v)
            pltpu.sync_copy(dst_hbm.at[pl.ds(base, cn)], dstc_v)
            pltpu.sync_copy(norm_hbm.at[pl.ds(base, cn)], nrmc_v)

            @pl.loop(0, cn // LANES, init_carry=cnt)
            def cnt2(g, cc):
                return do_group(g * LANES, cc)

            if cn % LANES:
                cnt2 = do_group(cn - LANES, cnt2,
                                mask_extra=lane >= LANES - (cn % LANES))
            return cnt2

        pltpu.sync_copy(cr_v, crid_hbm.at[pl.ds(wid * cap, cap)])
        pltpu.sync_copy(cd_v, cdst_hbm.at[pl.ds(wid * cap, cap)])
        pltpu.sync_copy(cw_v, cnorm_hbm.at[pl.ds(wid * cap, cap)])
        cnt_v[pl.ds(0, LANES)] = jnp.full((LANES,), total, jnp.int32)
        pltpu.sync_copy(cnt_v, ecnt_hbm.at[pl.ds(wid * LANES, LANES)])

    return k(tni, rid, dstn, norm)


def _msg_dyn(xra, xrb, crid, cdst, cnorm, ecnt, zrows, n_nodes, half, kk):
    nw = NC * NS
    cap = crid.shape[0] // nw
    rst = 624
    rww = n_nodes - rst * (NS - 1)
    mesh = plsc.VectorSubcoreMesh(core_axis_name="c", subcore_axis_name="s")

    @functools.partial(
        pl.kernel,
        out_type=[jax.ShapeDtypeStruct((n_nodes, half), jnp.float32)] * 2,
        mesh=mesh,
        compiler_params=pltpu.CompilerParams(needs_layout_passes=False),
        scratch_types=[
            pltpu.VMEM_SHARED((n_nodes, half), jnp.float32),
            pltpu.VMEM((kk,), jnp.int32),
            pltpu.VMEM((kk,), jnp.int32),
            pltpu.VMEM((kk,), jnp.float32),
            pltpu.VMEM((kk, half), jnp.float32),
            pltpu.VMEM((LANES,), jnp.int32),
            pltpu.SemaphoreType.DMA,
        ],
    )
    def k(xra_hbm, xrb_hbm, crid_hbm, cdst_hbm, cnorm_hbm, ecnt_hbm, z_hbm,
          outa_hbm, outb_hbm, agg_sh, rid_v, dst_v, nrm_v, msg_v, cnt_v, sem):
        c = lax.axis_index("c")
        s = lax.axis_index("s")

        def region(xr_hbm, w):
            pltpu.sync_copy(ecnt_hbm.at[pl.ds(w * LANES, LANES)], cnt_v)
            count = cnt_v[pl.ds(0, LANES)][0]

            @pl.loop(0, cap // kk)
            def _(kc):
                @pl.when(kc * kk < count)
                def _():
                    base = w * cap + kc * kk
                    pltpu.sync_copy(crid_hbm.at[pl.ds(base, kk)], rid_v)
                    pltpu.sync_copy(cdst_hbm.at[pl.ds(base, kk)], dst_v)
                    pltpu.sync_copy(cnorm_hbm.at[pl.ds(base, kk)], nrm_v)
                    pltpu.async_copy(xr_hbm.at[rid_v], msg_v, sem).wait()

                    @pl.loop(0, kk // LANES)
                    def _(g):
                        nvec = nrm_v[pl.ds(g * LANES, LANES)]
                        for i in range(LANES):
                            nv = nvec[i]
                            row = g * LANES + i
                            for j in range(half // LANES):
                                sl = pl.ds(j * LANES, LANES)
                                msg_v[row, sl] = msg_v[row, sl] * nv

                    pltpu.sync_copy(msg_v, agg_sh.at[dst_v], add=True)

        def half_body(xr_hbm, out_hbm):
            pltpu.sync_copy(z_hbm, agg_sh.at[pl.ds(s * rst, rww)])
            plsc.subcore_barrier()
            for wofs in range(NC):
                region(xr_hbm, s * NC + wofs)
            plsc.subcore_barrier()
            pltpu.sync_copy(agg_sh.at[pl.ds(s * rst, rww)],
                            out_hbm.at[pl.ds(s * rst, rww)])

        @pl.when(c == 0)
        def _():
            half_body(xra_hbm, outa_hbm)

        @pl.when(c == 1)
        def _():
            half_body(xrb_hbm, outb_hbm)

    return k(xra, xrb, crid, cdst, cnorm, ecnt, zrows)


def _gather_rows(table, idx, d):
    bp = idx.shape[0]
    bpw = bp // (NC * NS)
    mesh = plsc.VectorSubcoreMesh(core_axis_name="c", subcore_axis_name="s")

    @functools.partial(
        pl.kernel,
        out_type=jax.ShapeDtypeStruct((bp, d), jnp.float32),
        mesh=mesh,
        compiler_params=pltpu.CompilerParams(needs_layout_passes=False),
        scratch_types=[
            pltpu.VMEM((bpw,), jnp.int32),
            pltpu.VMEM((bpw, d), jnp.float32),
            pltpu.SemaphoreType.DMA,
        ],
    )
    def k(tab_hbm, idx_hbm, out_hbm, idx_v, rows_v, sem):
        wid = lax.axis_index("s") * NC + lax.axis_index("c")
        base = wid * bpw
        pltpu.sync_copy(idx_hbm.at[pl.ds(base, bpw)], idx_v)
        pltpu.async_copy(tab_hbm.at[idx_v], rows_v, sem).wait()
        pltpu.sync_copy(rows_v, out_hbm.at[pl.ds(base, bpw)])

    return k(table, idx)


def kernel(x, edge_index, edge_type, train_node_id, w1, r1, b1, w2, r2, b2,
           fcw, fcb):
    n, _ = x.shape
    r = w1.shape[0]
    dh = w1.shape[2]
    dout = w2.shape[2]
    bsz = train_node_id.shape[0]
    half = dh // 2

    src = edge_index[0]
    dst = edge_index[1]

    wcat1 = jnp.concatenate([w1, r1[None]], axis=0)
    wcat2 = jnp.concatenate([w2, r2[None]], axis=0)
    zrows = jnp.zeros((n - 624 * (NS - 1), half), jnp.float32)

    rid, norm = _edge_prep(src, dst, edge_type, n, r)

    e = edge_type.shape[0]
    kkc = 128
    rows = -(-(-(-e // kkc)) // (2 * NS)) * (2 * NS)
    ep = rows * kkc
    rid2 = jnp.pad(rid, (0, ep - e)).reshape(rows, kkc)
    dst2 = jnp.pad(dst, (0, ep - e)).reshape(rows, kkc)
    norm2 = jnp.pad(norm, (0, ep - e)).reshape(rows, kkc)

    wcat1b = wcat1.astype(jnp.bfloat16)
    wcat2b = wcat2.astype(jnp.bfloat16)
    xa, xb = _bmm_halves(x.astype(jnp.bfloat16), wcat1b, bn=1000)
    agga, aggb = _msg(xa.reshape((r + 1) * n, half),
                      xb.reshape((r + 1) * n, half),
                      rid2, dst2, norm2, zrows, n, half)
    h = _relu_add(agga, aggb, xa[r], xb[r], b1.reshape(1, dh), bn=1000)

    crid, cdstc, cnorm, ecnt = _edge_compact(train_node_id, rid, dst, norm,
                                              n, cap=5120)

    ya, yb = _bmm_halves(h.astype(jnp.bfloat16), wcat2b, bn=1000)
    agg2a, agg2b = _msg_dyn(ya.reshape((r + 1) * n, half),
                            yb.reshape((r + 1) * n, half),
                            crid, cdstc, cnorm, ecnt, zrows, n, half, kkc)
    h2 = _relu_add(agg2a, agg2b, ya[r], yb[r], b2.reshape(1, dout), bn=1000)

    bp = 1024
    tni = jnp.pad(train_node_id, (0, bp - bsz))
    hsel = _gather_rows(h2, tni, dout)
    fcwp = jnp.pad(fcw, ((0, 0), (0, 127)))
    fcbp = jnp.pad(fcb, (0, 127)).reshape(1, 128)
    y = _fc(hsel, fcwp, fcbp)
    return y[:bsz, 0]
